# baseline (speedup 1.0000x reference)
import math

import jax
import jax.numpy as jnp
from jax import lax
from jax.experimental import pallas as pl
from jax.experimental.pallas import tpu as pltpu
from jax.experimental.pallas import tpu_sc as plsc

D_MODEL_ = 64
SCALE_ = math.sqrt(D_MODEL_)

NC_ = 2
NS_ = 16
NW_ = NC_ * NS_
LANES_ = 16

CHUNK_ = 128
NBUF_ = 4
UNROLL_ = 4


def _make_lookup(batch, d_model):
    assert d_model % LANES_ == 0
    assert batch % (8 * NW_) == 0
    per_w = batch // NW_
    assert per_w % CHUNK_ == 0
    n_chunks = per_w // CHUNK_
    ng = n_chunks // NBUF_
    assert n_chunks % NBUF_ == 0 and ng >= 3
    assert CHUNK_ % UNROLL_ == 0
    nj = d_model // LANES_

    mesh = plsc.VectorSubcoreMesh(core_axis_name="c", subcore_axis_name="s")

    def body(x_hbm, table_hbm, out_hbm, idx_all, rows_v, outb_v, gsem, osem):
        wid = lax.axis_index("s") * NC_ + lax.axis_index("c")
        w_base = wid * per_w

        pltpu.sync_copy(x_hbm.at[pl.ds(w_base, per_w)], idx_all)

        def gather_start(g, b):
            pltpu.async_copy(
                table_hbm.at[idx_all.at[pl.ds(g * CHUNK_, CHUNK_)]],
                rows_v[b], gsem[b])

        def gather_wait(g, b):
            pltpu.make_async_copy(
                table_hbm.at[idx_all.at[pl.ds(g * CHUNK_, CHUNK_)]],
                rows_v[b], gsem[b]).wait()

        def scale(b):
            def srow(i, _):
                for u in range(UNROLL_):
                    r = i * UNROLL_ + u
                    for j in range(nj):
                        sl = pl.ds(j * LANES_, LANES_)
                        outb_v[b][r, sl] = rows_v[b][r, sl] * SCALE_
                return 0

            lax.fori_loop(0, CHUNK_ // UNROLL_, srow, 0)

        def out_start(g, b):
            pltpu.async_copy(
                outb_v[b], out_hbm.at[pl.ds(w_base + g * CHUNK_, CHUNK_)],
                osem[b])

        def out_wait(g, b):
            pltpu.make_async_copy(
                outb_v[b], out_hbm.at[pl.ds(w_base + g * CHUNK_, CHUNK_)],
                osem[b]).wait()

        for b in range(NBUF_):
            gather_start(b, b)
        for b in range(NBUF_):
            gather_wait(b, b)
            scale(b)
            out_start(b, b)
            gather_start(NBUF_ + b, b)

        def steady(i, _):
            for b in range(NBUF_):
                g = i * NBUF_ + b
                gather_wait(g, b)
                out_wait(g - NBUF_, b)
                scale(b)
                out_start(g, b)
                gather_start(g + NBUF_, b)
            return 0

        lax.fori_loop(1, ng - 1, steady, 0)

        for b in range(NBUF_):
            g = n_chunks - NBUF_ + b
            gather_wait(g, b)
            out_wait(g - NBUF_, b)
            scale(b)
            out_start(g, b)
        for b in range(NBUF_):
            out_wait(n_chunks - NBUF_ + b, b)

    return pl.kernel(
        body,
        out_type=jax.ShapeDtypeStruct((batch, d_model), jnp.float32),
        mesh=mesh,
        compiler_params=pltpu.CompilerParams(use_tc_tiling_on_sc=False),
        scratch_types=[
            pltpu.VMEM((per_w,), jnp.int32),
            [pltpu.VMEM((CHUNK_, d_model), jnp.float32)] * NBUF_,
            [pltpu.VMEM((CHUNK_, d_model), jnp.float32)] * NBUF_,
            [pltpu.SemaphoreType.DMA] * NBUF_,
            [pltpu.SemaphoreType.DMA] * NBUF_,
        ],
    )


def kernel(x, table):
    b0, b1 = x.shape
    d = table.shape[1]
    x_flat = x.reshape(b0 * b1).astype(jnp.int32)
    out = _make_lookup(b0 * b1, d)(x_flat, table)
    return out.reshape(b0, b1, d)


# speedup vs baseline: 1.0011x; 1.0011x over previous
import math

import jax
import jax.numpy as jnp
from jax import lax
from jax.experimental import pallas as pl
from jax.experimental.pallas import tpu as pltpu
from jax.experimental.pallas import tpu_sc as plsc

D_MODEL_ = 64
SCALE_ = math.sqrt(D_MODEL_)

NC_ = 2
NS_ = 16
NW_ = NC_ * NS_
LANES_ = 16

CHUNK_ = 128
NBUF_ = 4
UNROLL_ = 4


def _make_lookup(batch, d_model):
    assert d_model % LANES_ == 0
    assert batch % (8 * NW_) == 0
    per_w = batch // NW_
    assert per_w % CHUNK_ == 0
    n_chunks = per_w // CHUNK_
    ng = n_chunks // NBUF_
    assert n_chunks % NBUF_ == 0 and ng >= 3
    assert CHUNK_ % UNROLL_ == 0
    nj = d_model // LANES_

    mesh = plsc.VectorSubcoreMesh(core_axis_name="c", subcore_axis_name="s")

    def body(x_hbm, table_hbm, out_hbm, idx_all, rows_v, outb_v, gsem, osem):
        wid = lax.axis_index("s") * NC_ + lax.axis_index("c")
        w_base = wid * per_w

        pltpu.sync_copy(x_hbm.at[pl.ds(w_base, per_w)], idx_all)

        def gather_start(g, b):
            for k in range(CHUNK_ // LANES_):
                iv = idx_all[pl.ds(g * CHUNK_ + k * LANES_, LANES_)]
                pltpu.async_copy(
                    table_hbm.at[iv],
                    rows_v[b].at[pl.ds(k * LANES_, LANES_)], gsem[b])

        def gather_wait(g, b):
            pltpu.make_async_copy(
                out_hbm.at[pl.ds(w_base, CHUNK_)], rows_v[b], gsem[b]).wait()

        def scale(b):
            def srow(i, _):
                for u in range(UNROLL_):
                    r = i * UNROLL_ + u
                    for j in range(nj):
                        sl = pl.ds(j * LANES_, LANES_)
                        outb_v[b][r, sl] = rows_v[b][r, sl] * SCALE_
                return 0

            lax.fori_loop(0, CHUNK_ // UNROLL_, srow, 0)

        def out_start(g, b):
            pltpu.async_copy(
                outb_v[b], out_hbm.at[pl.ds(w_base + g * CHUNK_, CHUNK_)],
                osem[b])

        def out_wait(g, b):
            pltpu.make_async_copy(
                outb_v[b], out_hbm.at[pl.ds(w_base + g * CHUNK_, CHUNK_)],
                osem[b]).wait()

        for b in range(NBUF_):
            gather_start(b, b)
        for b in range(NBUF_):
            gather_wait(b, b)
            scale(b)
            out_start(b, b)
            gather_start(NBUF_ + b, b)

        def steady(i, _):
            for b in range(NBUF_):
                g = i * NBUF_ + b
                gather_wait(g, b)
                out_wait(g - NBUF_, b)
                scale(b)
                out_start(g, b)
                gather_start(g + NBUF_, b)
            return 0

        lax.fori_loop(1, ng - 1, steady, 0)

        for b in range(NBUF_):
            g = n_chunks - NBUF_ + b
            gather_wait(g, b)
            out_wait(g - NBUF_, b)
            scale(b)
            out_start(g, b)
        for b in range(NBUF_):
            out_wait(n_chunks - NBUF_ + b, b)

    return pl.kernel(
        body,
        out_type=jax.ShapeDtypeStruct((batch, d_model), jnp.float32),
        mesh=mesh,
        compiler_params=pltpu.CompilerParams(use_tc_tiling_on_sc=False),
        scratch_types=[
            pltpu.VMEM((per_w,), jnp.int32),
            [pltpu.VMEM((CHUNK_, d_model), jnp.float32)] * NBUF_,
            [pltpu.VMEM((CHUNK_, d_model), jnp.float32)] * NBUF_,
            [pltpu.SemaphoreType.DMA] * NBUF_,
            [pltpu.SemaphoreType.DMA] * NBUF_,
        ],
    )


def kernel(x, table):
    b0, b1 = x.shape
    d = table.shape[1]
    x_flat = x.reshape(b0 * b1).astype(jnp.int32)
    out = _make_lookup(b0 * b1, d)(x_flat, table)
    return out.reshape(b0, b1, d)
